# baseline (speedup 1.0000x reference)
import functools

import jax
import jax.numpy as jnp
from jax.experimental import pallas as pl
from jax.experimental.pallas import tpu as pltpu

_G = 4
_K = 160


def _vq_kernel(x_ref, cb_ref, zq_ref, codes_ref, loss_ref, *, n_total):
    b = pl.program_id(0)
    tt = pl.program_id(1)

    @pl.when(jnp.logical_and(b == 0, tt == 0))
    def _init():
        loss_ref[0, 0] = jnp.float32(0.0)

    x = x_ref[0]
    dg = cb_ref.shape[2]
    t_w = x.shape[1]
    dims = (((0,), (0,)), ((), ()))

    rowid = jax.lax.broadcasted_iota(jnp.int32, (8, _K), 0)
    iota_rows = jax.lax.broadcasted_iota(jnp.int32, (8, _K), 1).astype(jnp.float32)
    selw = jnp.where(rowid == 1, 1.0, iota_rows).astype(jnp.bfloat16)

    loss_tile = jnp.float32(0.0)
    for g in range(_G):
        xg = x[g * dg:(g + 1) * dg, :]
        cb = cb_ref[g]
        cb2 = jnp.sum(cb * cb, axis=1)
        x2 = jnp.sum(xg * xg, axis=0)
        m2 = jnp.dot((2.0 * cb).astype(jnp.bfloat16), xg.astype(jnp.bfloat16),
                     preferred_element_type=jnp.float32)
        d = (x2[None, :] + cb2[:, None]) - m2
        dmin = jnp.min(d, axis=0)
        of = (d == dmin[None, :])
        of_bf = of.astype(jnp.bfloat16)
        r = jax.lax.dot_general(selw, of_bf, (((1,), (0,)), ((), ())),
                                preferred_element_type=jnp.float32)
        cb_hi = cb.astype(jnp.bfloat16)
        cb_lo = (cb - cb_hi.astype(jnp.float32)).astype(jnp.bfloat16)
        zq_g = (jax.lax.dot_general(cb_hi, of_bf, dims,
                                    preferred_element_type=jnp.float32)
                + jax.lax.dot_general(cb_lo, of_bf, dims,
                                      preferred_element_type=jnp.float32))
        zq_ref[0, g * dg:(g + 1) * dg, :] = zq_g
        codes_ref[0, g, :] = r[0].astype(jnp.int32)

        @pl.when(jnp.max(r[1]) > 1.5)
        def _fix_ties(g=g, of=of, cb_hi=cb_hi, cb_lo=cb_lo):
            iota_k = jax.lax.broadcasted_iota(jnp.int32, (_K, t_w), 0)
            mi = jnp.min(jnp.where(of, iota_k, _K), axis=0)
            oh = (iota_k == mi[None, :]).astype(jnp.bfloat16)
            zq_s = (jax.lax.dot_general(cb_hi, oh, dims,
                                        preferred_element_type=jnp.float32)
                    + jax.lax.dot_general(cb_lo, oh, dims,
                                          preferred_element_type=jnp.float32))
            zq_ref[0, g * dg:(g + 1) * dg, :] = zq_s
            codes_ref[0, g, :] = mi.astype(jnp.int32)

        loss_tile += jnp.sum(dmin)

    loss_ref[0, 0] += loss_tile * (1.25 / n_total)


@jax.jit
def kernel(xin, codebooks):
    B, C, T = xin.shape
    G, K, dg = codebooks.shape
    t_blk = 1024 if T % 1024 == 0 else T
    grid = (B, T // t_blk)

    zq, codes, loss = pl.pallas_call(
        functools.partial(_vq_kernel, n_total=B * C * T),
        grid=grid,
        in_specs=[
            pl.BlockSpec((1, C, t_blk), lambda b, t: (b, 0, t)),
            pl.BlockSpec((G, K, dg), lambda b, t: (0, 0, 0)),
        ],
        out_specs=[
            pl.BlockSpec((1, C, t_blk), lambda b, t: (b, 0, t)),
            pl.BlockSpec((1, G, t_blk), lambda b, t: (b, 0, t)),
            pl.BlockSpec((1, 1), lambda b, t: (0, 0),
                         memory_space=pltpu.SMEM),
        ],
        out_shape=[
            jax.ShapeDtypeStruct((B, C, T), jnp.float32),
            jax.ShapeDtypeStruct((B, G, T), jnp.int32),
            jax.ShapeDtypeStruct((1, 1), jnp.float32),
        ],
    )(xin, codebooks)
    return zq, loss[0, 0], codes


# speedup vs baseline: 1.1597x; 1.1597x over previous
import functools

import jax
import jax.numpy as jnp
from jax.experimental import pallas as pl
from jax.experimental.pallas import tpu as pltpu

_G = 4
_K = 160


def _vq_kernel(x_ref, cbm_ref, cb2_ref, cbhi_ref, cblo_ref,
               zq_ref, codes_ref, loss_ref, *, n_total):
    b = pl.program_id(0)
    tt = pl.program_id(1)

    @pl.when(jnp.logical_and(b == 0, tt == 0))
    def _init():
        loss_ref[0, 0] = jnp.float32(0.0)

    x = x_ref[0]
    dg = cbhi_ref.shape[2]
    t_w = x.shape[1]
    dims = (((0,), (0,)), ((), ()))

    rowid = jax.lax.broadcasted_iota(jnp.int32, (8, _K), 0)
    iota_rows = jax.lax.broadcasted_iota(jnp.int32, (8, _K), 1).astype(jnp.float32)
    selw = jnp.where(rowid == 1, 1.0, iota_rows).astype(jnp.bfloat16)

    loss_tile = jnp.float32(0.0)
    for g in range(_G):
        xg = x[g * dg:(g + 1) * dg, :]
        cb2 = cb2_ref[0, g]
        x2 = jnp.sum(xg * xg, axis=0)
        m2 = jnp.dot(cbm_ref[g], xg.astype(jnp.bfloat16),
                     preferred_element_type=jnp.float32)
        d = (x2[None, :] + cb2[:, None]) - m2
        dmin = jnp.min(d, axis=0)
        of = (d == dmin[None, :])
        of_bf = of.astype(jnp.bfloat16)
        r = jax.lax.dot_general(selw, of_bf, (((1,), (0,)), ((), ())),
                                preferred_element_type=jnp.float32)
        zq_g = (jax.lax.dot_general(cbhi_ref[g], of_bf, dims,
                                    preferred_element_type=jnp.float32)
                + jax.lax.dot_general(cblo_ref[g], of_bf, dims,
                                      preferred_element_type=jnp.float32))
        zq_ref[0, g * dg:(g + 1) * dg, :] = zq_g
        codes_ref[0, g, :] = r[0].astype(jnp.int32)

        @pl.when(jnp.max(r[1]) > 1.5)
        def _fix_ties(g=g, of=of):
            iota_k = jax.lax.broadcasted_iota(jnp.int32, (_K, t_w), 0)
            mi = jnp.min(jnp.where(of, iota_k, _K), axis=0)
            oh = (iota_k == mi[None, :]).astype(jnp.bfloat16)
            zq_s = (jax.lax.dot_general(cbhi_ref[g], oh, dims,
                                        preferred_element_type=jnp.float32)
                    + jax.lax.dot_general(cblo_ref[g], oh, dims,
                                          preferred_element_type=jnp.float32))
            zq_ref[0, g * dg:(g + 1) * dg, :] = zq_s
            codes_ref[0, g, :] = mi.astype(jnp.int32)

        loss_tile += jnp.sum(dmin)

    loss_ref[0, 0] += loss_tile * (1.25 / n_total)


@jax.jit
def kernel(xin, codebooks):
    B, C, T = xin.shape
    G, K, dg = codebooks.shape
    t_blk = 2048 if T % 2048 == 0 else T
    grid = (B, T // t_blk)

    cbm = (2.0 * codebooks).astype(jnp.bfloat16)
    cb2 = jnp.sum(codebooks * codebooks, axis=2)[None]
    cbhi = codebooks.astype(jnp.bfloat16)
    cblo = (codebooks - cbhi.astype(jnp.float32)).astype(jnp.bfloat16)

    zq, codes, loss = pl.pallas_call(
        functools.partial(_vq_kernel, n_total=B * C * T),
        grid=grid,
        in_specs=[
            pl.BlockSpec((1, C, t_blk), lambda b, t: (b, 0, t)),
            pl.BlockSpec((G, K, dg), lambda b, t: (0, 0, 0)),
            pl.BlockSpec((1, G, K), lambda b, t: (0, 0, 0)),
            pl.BlockSpec((G, K, dg), lambda b, t: (0, 0, 0)),
            pl.BlockSpec((G, K, dg), lambda b, t: (0, 0, 0)),
        ],
        out_specs=[
            pl.BlockSpec((1, C, t_blk), lambda b, t: (b, 0, t)),
            pl.BlockSpec((1, G, t_blk), lambda b, t: (b, 0, t)),
            pl.BlockSpec((1, 1), lambda b, t: (0, 0),
                         memory_space=pltpu.SMEM),
        ],
        out_shape=[
            jax.ShapeDtypeStruct((B, C, T), jnp.float32),
            jax.ShapeDtypeStruct((B, G, T), jnp.int32),
            jax.ShapeDtypeStruct((1, 1), jnp.float32),
        ],
    )(xin, cbm, cb2, cbhi, cblo)
    return zq, loss[0, 0], codes
